# Initial kernel scaffold; baseline (speedup 1.0000x reference)
import jax
import jax.numpy as jnp
from jax.experimental import pallas as pl


def kernel(node_features, edge_indices, edge_features, enc_n_W1, enc_n_b1, enc_n_W2, enc_n_b2, enc_e_W1, enc_e_b1, enc_e_W2, enc_e_b2, mp_e_W1, mp_e_b1, mp_e_W2, mp_e_b2, mp_n_W1, mp_n_b1, mp_n_W2, mp_n_b2, dec_W1, dec_b1, dec_W2, dec_b2):
    raise NotImplementedError("write your pallas kernel here")


# speedup vs baseline: 8.2915x; 8.2915x over previous
import functools

import jax
import jax.numpy as jnp
import numpy as np
from jax import lax
from jax.experimental import pallas as pl
from jax.experimental.pallas import tpu as pltpu
from jax.experimental.pallas import tpu_sc as plsc

N = 10000
E = 160000
H = 128
OUT = 128
LAYERS = 6

NC = 2
NS = 16
NW = NC * NS
CH = 128
NCHUNK = E // CH
RPS = N // NS

NB = 5
BN = N // NB
BE = 2000
F32 = jnp.float32


def _dbf(x, w):
    return jnp.dot(x.astype(jnp.bfloat16), w.astype(jnp.bfloat16),
                   preferred_element_type=F32)


def _node_enc_body(nf, W1, b1, W2, b2, Wab, hn_o, a_o, b_o):
    h = jax.nn.gelu(_dbf(nf[...], W1[...]) + b1[...])
    hn = _dbf(h, W2[...]) + b2[...]
    hn_o[...] = hn
    ab = _dbf(hn, Wab[...])
    a_o[...] = ab[:, :H]
    b_o[...] = ab[:, H:]


def _node_enc(nf, W1, b1, W2, b2, Wab):
    full = pl.BlockSpec((H, H), lambda i: (0, 0))
    bias = pl.BlockSpec((1, H), lambda i: (0, 0))
    blk = pl.BlockSpec((BN, H), lambda i: (i, 0))
    return pl.pallas_call(
        _node_enc_body,
        grid=(NB,),
        in_specs=[blk, full, bias, full, bias,
                  pl.BlockSpec((H, 2 * H), lambda i: (0, 0))],
        out_specs=[blk, blk, blk],
        out_shape=[jax.ShapeDtypeStruct((N, H), F32)] * 3,
    )(nf, W1, b1, W2, b2, Wab)


def _edge_enc_body(ef, U1, c1, U2, c2, Wc, bc, *outs):
    q = jax.nn.gelu(_dbf(ef[...], U1[...]) + c1[...])
    he = _dbf(q, U2[...]) + c2[...]
    hp = _dbf(he, Wc[...]) + bc[...]
    for i in range(LAYERS):
        outs[i][...] = hp[:, i * H:(i + 1) * H]


def _edge_enc(ef, U1, c1, U2, c2, Wc, bc):
    blk = pl.BlockSpec((BE, H), lambda i: (i, 0))
    return pl.pallas_call(
        _edge_enc_body,
        grid=(E // BE,),
        in_specs=[pl.BlockSpec((BE, 16), lambda i: (i, 0)),
                  pl.BlockSpec((16, H), lambda i: (0, 0)),
                  pl.BlockSpec((1, H), lambda i: (0, 0)),
                  pl.BlockSpec((H, H), lambda i: (0, 0)),
                  pl.BlockSpec((1, H), lambda i: (0, 0)),
                  pl.BlockSpec((H, LAYERS * H), lambda i: (0, 0)),
                  pl.BlockSpec((1, LAYERS * H), lambda i: (0, 0))],
        out_specs=[blk] * LAYERS,
        out_shape=[jax.ShapeDtypeStruct((E, H), F32)] * LAYERS,
    )(ef, U1, c1, U2, c2, Wc, bc)


def _node_upd_body(hn, P, degP, W2, b2, Wn1, bn1, Wn2, bn2, Wab,
                   hn_o, a_o, b_o):
    p = P[0] + P[1]
    deg = degP[0, :, 0:1] + degP[1, :, 0:1]
    agg = jnp.dot(p, W2[...], preferred_element_type=F32,
                  precision=jax.lax.Precision.HIGHEST) + deg * b2[...]
    x = _dbf(jnp.concatenate([hn[...], agg], axis=1), Wn1[...]) + bn1[...]
    hn2 = hn[...] + _dbf(jax.nn.gelu(x), Wn2[...]) + bn2[...]
    hn_o[...] = hn2
    ab = _dbf(hn2, Wab[...])
    a_o[...] = ab[:, :H]
    b_o[...] = ab[:, H:]


def _final_body(hn, P, degP, W2, b2, Wn1, bn1, Wn2, bn2,
                dW1, db1, dW2, db2, out_o):
    p = P[0] + P[1]
    deg = degP[0, :, 0:1] + degP[1, :, 0:1]
    agg = jnp.dot(p, W2[...], preferred_element_type=F32,
                  precision=jax.lax.Precision.HIGHEST) + deg * b2[...]
    x = _dbf(jnp.concatenate([hn[...], agg], axis=1), Wn1[...]) + bn1[...]
    hn2 = hn[...] + _dbf(jax.nn.gelu(x), Wn2[...]) + bn2[...]
    d = jax.nn.gelu(_dbf(hn2, dW1[...]) + db1[...])
    out_o[...] = _dbf(d, dW2[...]) + db2[...]


_blkN = pl.BlockSpec((BN, H), lambda i: (i, 0))
_blkP = pl.BlockSpec((NC, BN, H), lambda i: (0, i, 0))
_blkD = pl.BlockSpec((NC, BN, 16), lambda i: (0, i, 0))
_full = pl.BlockSpec((H, H), lambda i: (0, 0))
_bias = pl.BlockSpec((1, H), lambda i: (0, 0))


def _node_upd(hn, P, degP, W2, b2, Wn1, bn1, Wn2, bn2, Wab):
    return pl.pallas_call(
        _node_upd_body,
        grid=(NB,),
        in_specs=[_blkN, _blkP, _blkD, _full, _bias,
                  pl.BlockSpec((2 * H, H), lambda i: (0, 0)), _bias,
                  _full, _bias, pl.BlockSpec((H, 2 * H), lambda i: (0, 0))],
        out_specs=[_blkN, _blkN, _blkN],
        out_shape=[jax.ShapeDtypeStruct((N, H), F32)] * 3,
    )(hn, P, degP, W2, b2, Wn1, bn1, Wn2, bn2, Wab)


def _final(hn, P, degP, W2, b2, Wn1, bn1, Wn2, bn2, dW1, db1, dW2, db2):
    return pl.pallas_call(
        _final_body,
        grid=(NB,),
        in_specs=[_blkN, _blkP, _blkD, _full, _bias,
                  pl.BlockSpec((2 * H, H), lambda i: (0, 0)), _bias,
                  _full, _bias, _full, _bias, _full,
                  pl.BlockSpec((1, OUT), lambda i: (0, 0))],
        out_specs=pl.BlockSpec((BN, OUT), lambda i: (i, 0)),
        out_shape=jax.ShapeDtypeStruct((N, OUT), F32),
    )(hn, P, degP, W2, b2, Wn1, bn1, Wn2, bn2, dW1, db1, dW2, db2)


@functools.lru_cache(maxsize=None)
def _sc_kernels():
    mesh = plsc.VectorSubcoreMesh(core_axis_name="c", subcore_axis_name="s",
                                  num_cores=NC, num_subcores=NS)
    cp = pltpu.CompilerParams(use_tc_tiling_on_sc=False)

    @functools.partial(
        pl.kernel,
        out_type=jax.ShapeDtypeStruct((NC, N, 16), F32),
        mesh=mesh,
        compiler_params=cp,
        scratch_types=[pltpu.VMEM((1, CH), jnp.int32),
                       pltpu.VMEM((CH, 16), F32),
                       pltpu.VMEM_SHARED((N, 16), F32)])
    def _deg_kernel(dst_hbm, ones_hbm, z16_hbm, out_hbm, dstv, onesv, acc):
        cid = lax.axis_index("c")
        sid = lax.axis_index("s")
        wid = cid * NS + sid
        pltpu.sync_copy(z16_hbm, acc.at[pl.ds(sid * RPS, RPS), :])
        pltpu.sync_copy(ones_hbm, onesv)
        plsc.subcore_barrier()
        nch = (NCHUNK - wid + NW - 1) // NW

        @pl.loop(0, nch)
        def _(j):
            off = (wid + j * NW) * CH
            pltpu.sync_copy(dst_hbm.at[pl.ds(off, CH)], dstv.at[0])
            pltpu.sync_copy(onesv, acc.at[dstv.at[0]], add=True)

        plsc.subcore_barrier()
        pltpu.sync_copy(acc.at[pl.ds(sid * RPS, RPS), :],
                        out_hbm.at[cid, pl.ds(sid * RPS, RPS), :])

    @functools.partial(
        pl.kernel,
        out_type=jax.ShapeDtypeStruct((NC, N, H), F32),
        mesh=mesh,
        compiler_params=cp,
        scratch_types=[pltpu.VMEM((1, CH), jnp.int32),
                       pltpu.VMEM((1, CH), jnp.int32),
                       pltpu.VMEM((CH, H), F32),
                       pltpu.VMEM((CH, H), F32),
                       pltpu.VMEM((CH, H), F32),
                       pltpu.VMEM_SHARED((N, H), F32)])
    def _edge_kernel(a_hbm, b_hbm, hp_hbm, src_hbm, dst_hbm, zn_hbm, out_hbm,
                     srcv, dstv, av, bv, tv, acc):
        cid = lax.axis_index("c")
        sid = lax.axis_index("s")
        wid = cid * NS + sid
        pltpu.sync_copy(zn_hbm, acc.at[pl.ds(sid * RPS, RPS), :])
        plsc.subcore_barrier()
        nch = (NCHUNK - wid + NW - 1) // NW

        @pl.loop(0, nch)
        def _(j):
            off = (wid + j * NW) * CH
            pltpu.sync_copy(src_hbm.at[pl.ds(off, CH)], srcv.at[0])
            pltpu.sync_copy(dst_hbm.at[pl.ds(off, CH)], dstv.at[0])
            pltpu.sync_copy(hp_hbm.at[pl.ds(off, CH), :], tv)
            pltpu.sync_copy(a_hbm.at[srcv.at[0]], av)
            pltpu.sync_copy(b_hbm.at[dstv.at[0]], bv)

            @pl.loop(0, CH)
            def _(r):
                for c in range(8):
                    sl = (pl.ds(r, 1), pl.ds(c * 16, 16))
                    x = av[sl] + bv[sl] + tv[sl]
                    u = np.float32(1.5957691216057308) * (
                        x + np.float32(0.044715) * (x * x * x))
                    g = x / (np.float32(1.0) + jnp.exp(-u))
                    tv[sl] = g.astype(jnp.bfloat16).astype(F32)

            pltpu.sync_copy(tv, acc.at[dstv.at[0]], add=True)

        plsc.subcore_barrier()
        pltpu.sync_copy(acc.at[pl.ds(sid * RPS, RPS), :],
                        out_hbm.at[cid, pl.ds(sid * RPS, RPS), :])

    return _deg_kernel, _edge_kernel


def kernel(node_features, edge_indices, edge_features,
           enc_n_W1, enc_n_b1, enc_n_W2, enc_n_b2,
           enc_e_W1, enc_e_b1, enc_e_W2, enc_e_b2,
           mp_e_W1, mp_e_b1, mp_e_W2, mp_e_b2,
           mp_n_W1, mp_n_b1, mp_n_W2, mp_n_b2,
           dec_W1, dec_b1, dec_W2, dec_b2):
    nf = node_features[0]
    src = edge_indices[0, :, 0]
    dst = edge_indices[0, :, 1]
    ef = edge_features[0]

    r1 = lambda v: v.reshape(1, -1)
    W1c_all = jnp.concatenate([mp_e_W1[i, 2 * H:, :] for i in range(LAYERS)],
                              axis=1)
    bc_all = mp_e_b1.reshape(1, LAYERS * H)
    Wab = [jnp.concatenate([mp_e_W1[i, :H, :], mp_e_W1[i, H:2 * H, :]], axis=1)
           for i in range(LAYERS)]

    zn = jnp.zeros((RPS, H), F32)
    z16 = jnp.zeros((RPS, 16), F32)
    ones16 = jnp.zeros((128, 16), F32).at[:, 0].set(1.0)

    _deg_kernel, _edge_kernel = _sc_kernels()
    hn, A, Bm = _node_enc(nf, enc_n_W1, r1(enc_n_b1), enc_n_W2, r1(enc_n_b2),
                          Wab[0])
    hps = _edge_enc(ef, enc_e_W1, r1(enc_e_b1), enc_e_W2, r1(enc_e_b2),
                    W1c_all, bc_all)
    degP = _deg_kernel(dst, ones16, z16)

    W2t = mp_e_W2.astype(jnp.bfloat16).astype(F32)
    for i in range(LAYERS):
        P = _edge_kernel(A, Bm, hps[i], src, dst, zn)
        if i < LAYERS - 1:
            hn, A, Bm = _node_upd(
                hn, P, degP, W2t[i], r1(mp_e_b2[i]),
                mp_n_W1[i], r1(mp_n_b1[i]),
                mp_n_W2[i], r1(mp_n_b2[i]), Wab[i + 1])
        else:
            out = _final(
                hn, P, degP, W2t[i], r1(mp_e_b2[i]),
                mp_n_W1[i], r1(mp_n_b1[i]),
                mp_n_W2[i], r1(mp_n_b2[i]),
                dec_W1, r1(dec_b1), dec_W2, r1(dec_b2))
    return out.reshape(1, N, OUT)
